# baseline (speedup 1.0000x reference)
import functools

import jax
import jax.numpy as jnp
from jax.experimental import pallas as pl
from jax.experimental.pallas import tpu as pltpu
from jax.experimental.pallas import tpu_sc as plsc

_B = 8
_D = 1024

_mesh = plsc.ScalarSubcoreMesh(axis_name="c", num_cores=1)


@functools.partial(
    pl.kernel,
    mesh=_mesh,
    out_type=jax.ShapeDtypeStruct((_B, 1, _D), jnp.float32),
)
def _write_narrowed(token_hbm, out_hbm):
    pltpu.sync_copy(token_hbm, out_hbm)


def kernel(input_token, input_pos, kv_cache):
    del input_pos, kv_cache
    return _write_narrowed(input_token)
